# baseline (speedup 1.0000x reference)
import jax
import jax.numpy as jnp
from jax import lax
from jax.experimental import pallas as pl
from jax.experimental.pallas import tpu as pltpu
from jax.experimental.pallas import tpu_sc as plsc

_N = 16384
_D = 3
_NC = 2
_NS = 16
_L = 16
_NW = _NS
_PER = _N // _NW
_STEPS = _PER // _L


_H = _PER // 2


def _router_body(labels_hbm, tab_t_hbm, out_t_hbm, lab_v, tab_v, out_v,
                 sem_l0, sem_l1, sem_t, sem_o):
    wid = lax.axis_index("s")
    base = wid * _PER
    cp_l0 = pltpu.async_copy(labels_hbm.at[pl.ds(base, _H)],
                             lab_v.at[pl.ds(0, _H)], sem_l0)
    cp_l1 = pltpu.async_copy(labels_hbm.at[pl.ds(base + _H, _H)],
                             lab_v.at[pl.ds(_H, _H)], sem_l1)
    cp_t = pltpu.async_copy(tab_t_hbm, tab_v, sem_t)
    cp_l0.wait()
    cp_t.wait()

    @plsc.parallel_loop(0, _STEPS // 2, unroll=4)
    def step0(i):
        lab = lab_v[pl.ds(i * _L, _L)]
        for j in range(_D):
            vals = plsc.load_gather(tab_v.at[j], [lab])
            out_v[j, pl.ds(i * _L, _L)] = vals

    cp_o0 = pltpu.async_copy(out_v.at[:, pl.ds(0, _H)],
                             out_t_hbm.at[:, pl.ds(base, _H)], sem_o)
    cp_l1.wait()

    @plsc.parallel_loop(_STEPS // 2, _STEPS, unroll=4)
    def step1(i):
        lab = lab_v[pl.ds(i * _L, _L)]
        for j in range(_D):
            vals = plsc.load_gather(tab_v.at[j], [lab])
            out_v[j, pl.ds(i * _L, _L)] = vals

    pltpu.sync_copy(out_v.at[:, pl.ds(_H, _H)],
                    out_t_hbm.at[:, pl.ds(base + _H, _H)])
    cp_o0.wait()


def kernel(x, comb_attr_labels, mask_arr):
    labels = comb_attr_labels.astype(jnp.int32)
    run = pl.kernel(
        _router_body,
        mesh=plsc.VectorSubcoreMesh(core_axis_name="c", subcore_axis_name="s", num_cores=1),
        compiler_params=pltpu.CompilerParams(needs_layout_passes=False),
        out_type=jax.ShapeDtypeStruct((_D, _N), jnp.float32),
        scratch_types=[
            pltpu.VMEM((_PER,), jnp.int32),
            pltpu.VMEM((_D, 8), jnp.float32),
            pltpu.VMEM((_D, _PER), jnp.float32),
            pltpu.SemaphoreType.DMA,
            pltpu.SemaphoreType.DMA,
            pltpu.SemaphoreType.DMA,
            pltpu.SemaphoreType.DMA,
        ],
    )
    out_t = run(labels, mask_arr.T)
    return out_t.T


# speedup vs baseline: 1.0063x; 1.0063x over previous
import jax
import jax.numpy as jnp
from jax import lax
from jax.experimental import pallas as pl
from jax.experimental.pallas import tpu as pltpu
from jax.experimental.pallas import tpu_sc as plsc

_N = 16384
_D = 3
_NC = 2
_NS = 16
_L = 16
_NW = _NS
_PER = _N // _NW
_STEPS = _PER // _L


def _router_body(labels_hbm, tab_t_hbm, out_t_hbm, lab_v, tab_v, out_v,
                 sem_l, sem_t):
    wid = lax.axis_index("s")
    base = wid * _PER
    cp_l = pltpu.async_copy(labels_hbm.at[pl.ds(base, _PER)], lab_v, sem_l)
    cp_t = pltpu.async_copy(tab_t_hbm, tab_v, sem_t)
    cp_l.wait()
    cp_t.wait()
    @plsc.parallel_loop(0, _STEPS, unroll=4)
    def step(i):
        lab = lab_v[pl.ds(i * _L, _L)]
        for j in range(_D):
            vals = plsc.load_gather(tab_v.at[j], [lab])
            out_v[j, pl.ds(i * _L, _L)] = vals
    pltpu.sync_copy(out_v, out_t_hbm.at[:, pl.ds(base, _PER)])


def kernel(x, comb_attr_labels, mask_arr):
    labels = comb_attr_labels.astype(jnp.int32)
    run = pl.kernel(
        _router_body,
        mesh=plsc.VectorSubcoreMesh(core_axis_name="c", subcore_axis_name="s", num_cores=1),
        compiler_params=pltpu.CompilerParams(needs_layout_passes=False),
        out_type=jax.ShapeDtypeStruct((_D, _N), jnp.float32),
        scratch_types=[
            pltpu.VMEM((_PER,), jnp.int32),
            pltpu.VMEM((_D, 8), jnp.float32),
            pltpu.VMEM((_D, _PER), jnp.float32),
            pltpu.SemaphoreType.DMA,
            pltpu.SemaphoreType.DMA,
        ],
    )
    out_t = run(labels, mask_arr.T)
    return out_t.T


# speedup vs baseline: 1.0073x; 1.0010x over previous
import jax
import jax.numpy as jnp
from jax import lax
from jax.experimental import pallas as pl
from jax.experimental.pallas import tpu as pltpu
from jax.experimental.pallas import tpu_sc as plsc

_N = 16384
_D = 3
_NS = 16
_L = 16
_NW = _NS
_PER = _N // _NW
_STEPS = _PER // _L


def _router_body(labels_hbm, tab_t_hbm, out_t_hbm, lab_v, tab_v, out_v,
                 sem_l, sem_t):
    wid = lax.axis_index("s")
    base = wid * _PER
    cp_l = pltpu.async_copy(labels_hbm.at[pl.ds(base, _PER)], lab_v, sem_l)
    cp_t = pltpu.async_copy(tab_t_hbm, tab_v, sem_t)
    cp_l.wait()
    cp_t.wait()

    @plsc.parallel_loop(0, _STEPS, unroll=4)
    def step(i):
        lab = lab_v[pl.ds(i * _L, _L)]
        for j in range(_D):
            vals = plsc.load_gather(tab_v.at[j], [lab])
            out_v[j, pl.ds(i * _L, _L)] = vals

    pltpu.sync_copy(out_v, out_t_hbm.at[:, pl.ds(base, _PER)])


def kernel(x, comb_attr_labels, mask_arr):
    labels = comb_attr_labels.astype(jnp.int32)
    run = pl.kernel(
        _router_body,
        mesh=plsc.VectorSubcoreMesh(core_axis_name="c", subcore_axis_name="s",
                                    num_cores=1),
        compiler_params=pltpu.CompilerParams(needs_layout_passes=False),
        out_type=jax.ShapeDtypeStruct((_D, _N), jnp.float32),
        scratch_types=[
            pltpu.VMEM((_PER,), jnp.int32),
            pltpu.VMEM((_D, 8), jnp.float32),
            pltpu.VMEM((_D, _PER), jnp.float32),
            pltpu.SemaphoreType.DMA,
            pltpu.SemaphoreType.DMA,
        ],
    )
    out_t = run(labels, mask_arr.T)
    return out_t.T
